# baseline (speedup 1.0000x reference)
import jax
import jax.numpy as jnp
from jax.experimental import pallas as pl
from jax.experimental.pallas import tpu as pltpu

_K = 50
_BLOCK_R = 256


_T = 8
_NC = 64
_NEG = -3.4e38


def _emit_outputs(acc_s, acc_i, out_s_ref, out_i_ref):
    ts = acc_s[:, :_K]
    out_s_ref[...] = jnp.where(ts < -1e29, -jnp.inf, ts)
    out_i_ref[...] = acc_i[:, :_K]


def _naive_topk(s_ref, col, n, r, colk, out_s_ref, out_i_ref):
    def body(k, carry):
        acc_s, acc_i = carry
        cur = s_ref[...]
        m = jnp.max(cur, axis=1)
        hit = cur == m[:, None]
        idx = jnp.min(jnp.where(hit, col, n), axis=1)
        s_ref[...] = jnp.where(col == idx[:, None], _NEG, cur)
        acc_s = jnp.where(colk == k, m[:, None], acc_s)
        acc_i = jnp.where(colk == k, idx[:, None], acc_i)
        return acc_s, acc_i

    acc_s, acc_i = jax.lax.fori_loop(
        0, _K, body,
        (jnp.zeros((r, 64), jnp.float32), jnp.zeros((r, 64), jnp.int32)),
    )
    _emit_outputs(acc_s, acc_i, out_s_ref, out_i_ref)


def _score_topk_body_inner(m_blk, wt_ref, b_ref, mt_ref,
                           out_s_ref, out_i_ref, s_ref, cand_ref, gidx_ref, r0):
    r = m_blk.shape[0]
    n = mt_ref.shape[1]
    pid = r0 + pl.program_id(0)

    proj = jnp.dot(m_blk[...], wt_ref[...], preferred_element_type=jnp.float32)
    proj = proj + b_ref[...]
    s = jnp.dot(proj, mt_ref[...], preferred_element_type=jnp.float32)

    col = jax.lax.broadcasted_iota(jnp.int32, (r, n), 1)
    row = pid * r + jax.lax.broadcasted_iota(jnp.int32, (r, n), 0)
    neg = -1e30 - col.astype(jnp.float32) * 1e24
    s = jnp.where(col < row, s, neg)
    s_ref[...] = s

    colk = jax.lax.broadcasted_iota(jnp.int32, (r, 64), 1)
    lane = jax.lax.broadcasted_iota(jnp.int32, (r, 128), 1)
    lane64 = jax.lax.broadcasted_iota(jnp.int32, (r, _NC), 1)
    half = (lane >= _NC).astype(jnp.int32)
    vc = n // 128

    vals = [s[:, v * 128:(v + 1) * 128] for v in range(vc)]
    cvs, cps = [], []
    prev_p = None
    for t in range(_T):
        if t > 0:
            pe = jnp.concatenate([prev_p, prev_p], axis=1)
            vals = [jnp.where(pe == half + 2 * v, _NEG, vals[v])
                    for v in range(vc)]
        cm = vals[0]
        pp = half
        for v in range(1, vc):
            upd = vals[v] > cm
            cm = jnp.where(upd, vals[v], cm)
            pp = jnp.where(upd, half + 2 * v, pp)
        cma, cmb = cm[:, :_NC], cm[:, _NC:]
        ppa, ppb = pp[:, :_NC], pp[:, _NC:]
        take = (cmb > cma) | ((cmb == cma) & (ppb < ppa))
        cvs.append(jnp.where(take, cmb, cma))
        cps.append(jnp.where(take, ppb, ppa))
        prev_p = cps[-1]

    cand_ref[...] = jnp.concatenate(cvs, axis=1)
    gidx_ref[...] = jnp.concatenate(
        [cps[t] * _NC + lane64 for t in range(_T)], axis=1)

    _S = 5

    def ext_body(j, carry):
        acc_s, acc_i = carry
        c = cand_ref[...]
        g = gidx_ref[...]
        big = jnp.int32(1 << 30)
        for u in range(_S):
            k = j * _S + u
            mrow = jnp.max(c, axis=1, keepdims=True)
            gi = jnp.min(jnp.where(c == mrow, g, big), axis=1, keepdims=True)
            c = jnp.where(g == gi, _NEG, c)
            acc_s = jnp.where(colk == k, mrow, acc_s)
            acc_i = jnp.where(colk == k, gi, acc_i)
        cand_ref[...] = c
        return acc_s, acc_i

    acc_s, acc_i = jax.lax.fori_loop(
        0, _K // _S, ext_body,
        (jnp.zeros((r, 64), jnp.float32), jnp.zeros((r, 64), jnp.int32)),
    )
    _emit_outputs(acc_s, acc_i, out_s_ref, out_i_ref)
    bad = cand_ref[:, (_T - 1) * _NC:] == _NEG

    @pl.when(jnp.any(bad))
    def _():
        _naive_topk(s_ref, col, n, r, colk, out_s_ref, out_i_ref)


def _run_span(mentions, wt, b2, mt, row_start, n_rows, width):
    n, f = mentions.shape
    blk = min(_BLOCK_R, n_rows)
    r0 = row_start // blk

    def body(m_blk, wt_ref, b_ref, mt_ref, out_s_ref, out_i_ref,
             s_ref, cand_ref, gidx_ref):
        _score_topk_body_inner(m_blk, wt_ref, b_ref, mt_ref,
                               out_s_ref, out_i_ref, s_ref,
                               cand_ref, gidx_ref, r0)

    return pl.pallas_call(
        body,
        grid=(n_rows // blk,),
        in_specs=[
            pl.BlockSpec((blk, f), lambda i: (r0 + i, 0)),
            pl.BlockSpec((f, f), lambda i: (0, 0)),
            pl.BlockSpec((1, f), lambda i: (0, 0)),
            pl.BlockSpec((f, width), lambda i: (0, 0)),
        ],
        out_specs=[
            pl.BlockSpec((blk, _K), lambda i: (i, 0)),
            pl.BlockSpec((blk, _K), lambda i: (i, 0)),
        ],
        out_shape=[
            jax.ShapeDtypeStruct((n_rows, _K), jnp.float32),
            jax.ShapeDtypeStruct((n_rows, _K), jnp.int32),
        ],
        scratch_shapes=[pltpu.VMEM((blk, width), jnp.float32),
                        pltpu.VMEM((blk, _T * _NC), jnp.float32),
                        pltpu.VMEM((blk, _T * _NC), jnp.int32)],
    )(mentions, wt, b2, mt[:, :width])


def _run_span_tuple(*args, **kw):
    out = _run_span(*args, **kw)
    return out[0], out[1]


def kernel(mentions, W, b):
    n, f = mentions.shape
    wt = W.T
    mt = mentions.T
    b2 = b.reshape(1, f)
    if n <= 1024:
        return _run_span_tuple(mentions, wt, b2, mt, 0, n, n)
    n_span = 8
    span = n // n_span
    parts = [
        _run_span(mentions, wt, b2, mt, k * span, span, (k + 1) * span)
        for k in range(n_span)
    ]
    out_s = jnp.concatenate([p[0] for p in parts], axis=0)
    out_i = jnp.concatenate([p[1] for p in parts], axis=0)
    return out_s, out_i


# speedup vs baseline: 1.1608x; 1.1608x over previous
import jax
import jax.numpy as jnp
from jax.experimental import pallas as pl
from jax.experimental.pallas import tpu as pltpu

_K = 50
_BLOCK_R = 256


_T = 8
_NC = 64
_NEG = -3.4e38


def _emit_outputs(acc_s, acc_i, out_s_ref, out_i_ref):
    ts = acc_s[:, :_K]
    out_s_ref[...] = jnp.where(ts < -1e29, -jnp.inf, ts)
    out_i_ref[...] = acc_i[:, :_K]


def _naive_topk(s_ref, col, n, r, colk, out_s_ref, out_i_ref):
    def body(k, carry):
        acc_s, acc_i = carry
        cur = s_ref[...]
        m = jnp.max(cur, axis=1)
        hit = cur == m[:, None]
        idx = jnp.min(jnp.where(hit, col, n), axis=1)
        s_ref[...] = jnp.where(col == idx[:, None], _NEG, cur)
        acc_s = jnp.where(colk == k, m[:, None], acc_s)
        acc_i = jnp.where(colk == k, idx[:, None], acc_i)
        return acc_s, acc_i

    acc_s, acc_i = jax.lax.fori_loop(
        0, _K, body,
        (jnp.zeros((r, 64), jnp.float32), jnp.zeros((r, 64), jnp.int32)),
    )
    _emit_outputs(acc_s, acc_i, out_s_ref, out_i_ref)


def _score_topk_body_inner(m_blk, wt_ref, b_ref, mt_ref,
                           out_s_ref, out_i_ref, s_ref, cand_ref, gidx_ref, r0):
    r = m_blk.shape[0]
    n = mt_ref.shape[1]
    pid = r0 + pl.program_id(0)

    proj = jnp.dot(m_blk[...], wt_ref[...], preferred_element_type=jnp.float32)
    proj = proj + b_ref[...]
    s = jnp.dot(proj, mt_ref[...], preferred_element_type=jnp.float32)

    col = jax.lax.broadcasted_iota(jnp.int32, (r, n), 1)
    row = pid * r + jax.lax.broadcasted_iota(jnp.int32, (r, n), 0)
    neg = -1e30 - col.astype(jnp.float32) * 1e24
    s = jnp.where(col < row, s, neg)
    s_ref[...] = s

    colk = jax.lax.broadcasted_iota(jnp.int32, (r, 64), 1)
    lane = jax.lax.broadcasted_iota(jnp.int32, (r, 128), 1)
    lane64 = jax.lax.broadcasted_iota(jnp.int32, (r, _NC), 1)
    half = (lane >= _NC).astype(jnp.int32)
    vc = n // 128

    vals = [s[:, v * 128:(v + 1) * 128] for v in range(vc)]
    cvs, cps = [], []
    prev_p = None
    for t in range(_T):
        if t > 0:
            pe = jnp.concatenate([prev_p, prev_p], axis=1)
            vals = [jnp.where(pe == half + 2 * v, _NEG, vals[v])
                    for v in range(vc)]
        cm = vals[0]
        pp = half
        for v in range(1, vc):
            upd = vals[v] > cm
            cm = jnp.where(upd, vals[v], cm)
            pp = jnp.where(upd, half + 2 * v, pp)
        cma, cmb = cm[:, :_NC], cm[:, _NC:]
        ppa, ppb = pp[:, :_NC], pp[:, _NC:]
        take = (cmb > cma) | ((cmb == cma) & (ppb < ppa))
        cvs.append(jnp.where(take, cmb, cma))
        cps.append(jnp.where(take, ppb, ppa))
        prev_p = cps[-1]

    cand_ref[...] = jnp.concatenate(cvs, axis=1)
    gidx_ref[...] = jnp.concatenate(
        [cps[t] * _NC + lane64 for t in range(_T)], axis=1)

    _S = 10

    def ext_body(j, carry):
        acc_s, acc_i = carry
        c = cand_ref[...]
        g = gidx_ref[...]
        big = jnp.int32(1 << 30)
        for u in range(_S):
            k = j * _S + u
            mrow = jnp.max(c, axis=1, keepdims=True)
            gi = jnp.min(jnp.where(c == mrow, g, big), axis=1, keepdims=True)
            c = jnp.where(g == gi, _NEG, c)
            acc_s = jnp.where(colk == k, mrow, acc_s)
            acc_i = jnp.where(colk == k, gi, acc_i)
        cand_ref[...] = c
        return acc_s, acc_i

    acc_s, acc_i = jax.lax.fori_loop(
        0, _K // _S, ext_body,
        (jnp.zeros((r, 64), jnp.float32), jnp.zeros((r, 64), jnp.int32)),
    )
    _emit_outputs(acc_s, acc_i, out_s_ref, out_i_ref)
    bad = cand_ref[:, (_T - 1) * _NC:] == _NEG

    @pl.when(jnp.any(bad))
    def _():
        _naive_topk(s_ref, col, n, r, colk, out_s_ref, out_i_ref)


def _run_span(mentions, wt, b2, mt, row_start, n_rows, width):
    n, f = mentions.shape
    blk = min(_BLOCK_R, n_rows)
    r0 = row_start // blk

    def body(m_blk, wt_ref, b_ref, mt_ref, out_s_ref, out_i_ref,
             s_ref, cand_ref, gidx_ref):
        _score_topk_body_inner(m_blk, wt_ref, b_ref, mt_ref,
                               out_s_ref, out_i_ref, s_ref,
                               cand_ref, gidx_ref, r0)

    return pl.pallas_call(
        body,
        grid=(n_rows // blk,),
        in_specs=[
            pl.BlockSpec((blk, f), lambda i: (r0 + i, 0)),
            pl.BlockSpec((f, f), lambda i: (0, 0)),
            pl.BlockSpec((1, f), lambda i: (0, 0)),
            pl.BlockSpec((f, width), lambda i: (0, 0)),
        ],
        out_specs=[
            pl.BlockSpec((blk, _K), lambda i: (i, 0)),
            pl.BlockSpec((blk, _K), lambda i: (i, 0)),
        ],
        out_shape=[
            jax.ShapeDtypeStruct((n_rows, _K), jnp.float32),
            jax.ShapeDtypeStruct((n_rows, _K), jnp.int32),
        ],
        scratch_shapes=[pltpu.VMEM((blk, width), jnp.float32),
                        pltpu.VMEM((blk, _T * _NC), jnp.float32),
                        pltpu.VMEM((blk, _T * _NC), jnp.int32)],
    )(mentions, wt, b2, mt[:, :width])


def _run_span_tuple(*args, **kw):
    out = _run_span(*args, **kw)
    return out[0], out[1]


def kernel(mentions, W, b):
    n, f = mentions.shape
    wt = W.T
    mt = mentions.T
    b2 = b.reshape(1, f)
    if n <= 1024:
        return _run_span_tuple(mentions, wt, b2, mt, 0, n, n)
    n_span = 4
    span = n // n_span
    parts = [
        _run_span(mentions, wt, b2, mt, k * span, span, (k + 1) * span)
        for k in range(n_span)
    ]
    out_s = jnp.concatenate([p[0] for p in parts], axis=0)
    out_i = jnp.concatenate([p[1] for p in parts], axis=0)
    return out_s, out_i


# speedup vs baseline: 1.1721x; 1.0097x over previous
import jax
import jax.numpy as jnp
from jax.experimental import pallas as pl
from jax.experimental.pallas import tpu as pltpu

_K = 50
_BLOCK_R = 256


_T = 8
_NC = 64
_NEG = -3.4e38


def _emit_outputs(acc_s, acc_i, out_s_ref, out_i_ref):
    ts = acc_s[:, :_K]
    out_s_ref[...] = jnp.where(ts < -1e29, -jnp.inf, ts)
    out_i_ref[...] = acc_i[:, :_K]


def _naive_topk(s_ref, col, n, r, colk, out_s_ref, out_i_ref):
    def body(k, carry):
        acc_s, acc_i = carry
        cur = s_ref[...]
        m = jnp.max(cur, axis=1)
        hit = cur == m[:, None]
        idx = jnp.min(jnp.where(hit, col, n), axis=1)
        s_ref[...] = jnp.where(col == idx[:, None], _NEG, cur)
        acc_s = jnp.where(colk == k, m[:, None], acc_s)
        acc_i = jnp.where(colk == k, idx[:, None], acc_i)
        return acc_s, acc_i

    acc_s, acc_i = jax.lax.fori_loop(
        0, _K, body,
        (jnp.zeros((r, 64), jnp.float32), jnp.zeros((r, 64), jnp.int32)),
    )
    _emit_outputs(acc_s, acc_i, out_s_ref, out_i_ref)


def _score_topk_body_inner(m_blk, wt_ref, b_ref, mt_ref,
                           out_s_ref, out_i_ref, s_ref, cand_ref, gidx_ref, r0):
    r = m_blk.shape[0]
    n = mt_ref.shape[1]
    pid = r0 + pl.program_id(0)

    proj = jnp.dot(m_blk[...], wt_ref[...], preferred_element_type=jnp.float32)
    proj = proj + b_ref[...]
    s = jnp.dot(proj, mt_ref[...], preferred_element_type=jnp.float32)

    col = jax.lax.broadcasted_iota(jnp.int32, (r, n), 1)
    row = pid * r + jax.lax.broadcasted_iota(jnp.int32, (r, n), 0)
    neg = -1e30 - col.astype(jnp.float32) * 1e24
    s = jnp.where(col < row, s, neg)
    s_ref[...] = s

    colk = jax.lax.broadcasted_iota(jnp.int32, (r, 64), 1)
    lane = jax.lax.broadcasted_iota(jnp.int32, (r, 128), 1)
    lane64 = jax.lax.broadcasted_iota(jnp.int32, (r, _NC), 1)
    half = (lane >= _NC).astype(jnp.int32)
    vc = n // 128

    vals = [s[:, v * 128:(v + 1) * 128] for v in range(vc)]
    cvs, cps = [], []
    prev_p = None
    for t in range(_T):
        if t > 0:
            pe = jnp.concatenate([prev_p, prev_p], axis=1)
            vals = [jnp.where(pe == half + 2 * v, _NEG, vals[v])
                    for v in range(vc)]
        cm = vals[0]
        pp = half
        for v in range(1, vc):
            upd = vals[v] > cm
            cm = jnp.where(upd, vals[v], cm)
            pp = jnp.where(upd, half + 2 * v, pp)
        cma, cmb = cm[:, :_NC], cm[:, _NC:]
        ppa, ppb = pp[:, :_NC], pp[:, _NC:]
        take = (cmb > cma) | ((cmb == cma) & (ppb < ppa))
        cvs.append(jnp.where(take, cmb, cma))
        cps.append(jnp.where(take, ppb, ppa))
        prev_p = cps[-1]

    cand_ref[...] = jnp.concatenate(cvs, axis=1)
    gidx_ref[...] = jnp.concatenate(
        [cps[t] * _NC + lane64 for t in range(_T)], axis=1)

    _S = 25

    def ext_body(j, carry):
        acc_s, acc_i = carry
        c = cand_ref[...]
        g = gidx_ref[...]
        big = jnp.int32(1 << 30)
        for u in range(_S):
            k = j * _S + u
            mrow = jnp.max(c, axis=1, keepdims=True)
            gi = jnp.min(jnp.where(c == mrow, g, big), axis=1, keepdims=True)
            c = jnp.where(g == gi, _NEG, c)
            acc_s = jnp.where(colk == k, mrow, acc_s)
            acc_i = jnp.where(colk == k, gi, acc_i)
        cand_ref[...] = c
        return acc_s, acc_i

    acc_s, acc_i = jax.lax.fori_loop(
        0, _K // _S, ext_body,
        (jnp.zeros((r, 64), jnp.float32), jnp.zeros((r, 64), jnp.int32)),
    )
    _emit_outputs(acc_s, acc_i, out_s_ref, out_i_ref)
    bad = cand_ref[:, (_T - 1) * _NC:] == _NEG

    @pl.when(jnp.any(bad))
    def _():
        _naive_topk(s_ref, col, n, r, colk, out_s_ref, out_i_ref)


def _run_span(mentions, wt, b2, mt, row_start, n_rows, width):
    n, f = mentions.shape
    blk = min(_BLOCK_R, n_rows)
    r0 = row_start // blk

    def body(m_blk, wt_ref, b_ref, mt_ref, out_s_ref, out_i_ref,
             s_ref, cand_ref, gidx_ref):
        _score_topk_body_inner(m_blk, wt_ref, b_ref, mt_ref,
                               out_s_ref, out_i_ref, s_ref,
                               cand_ref, gidx_ref, r0)

    return pl.pallas_call(
        body,
        grid=(n_rows // blk,),
        in_specs=[
            pl.BlockSpec((blk, f), lambda i: (r0 + i, 0)),
            pl.BlockSpec((f, f), lambda i: (0, 0)),
            pl.BlockSpec((1, f), lambda i: (0, 0)),
            pl.BlockSpec((f, width), lambda i: (0, 0)),
        ],
        out_specs=[
            pl.BlockSpec((blk, _K), lambda i: (i, 0)),
            pl.BlockSpec((blk, _K), lambda i: (i, 0)),
        ],
        out_shape=[
            jax.ShapeDtypeStruct((n_rows, _K), jnp.float32),
            jax.ShapeDtypeStruct((n_rows, _K), jnp.int32),
        ],
        scratch_shapes=[pltpu.VMEM((blk, width), jnp.float32),
                        pltpu.VMEM((blk, _T * _NC), jnp.float32),
                        pltpu.VMEM((blk, _T * _NC), jnp.int32)],
    )(mentions, wt, b2, mt[:, :width])


def _run_span_tuple(*args, **kw):
    out = _run_span(*args, **kw)
    return out[0], out[1]


def kernel(mentions, W, b):
    n, f = mentions.shape
    wt = W.T
    mt = mentions.T
    b2 = b.reshape(1, f)
    if n <= 1024:
        return _run_span_tuple(mentions, wt, b2, mt, 0, n, n)
    n_span = 4
    span = n // n_span
    parts = [
        _run_span(mentions, wt, b2, mt, k * span, span, (k + 1) * span)
        for k in range(n_span)
    ]
    out_s = jnp.concatenate([p[0] for p in parts], axis=0)
    out_i = jnp.concatenate([p[1] for p in parts], axis=0)
    return out_s, out_i
